# baseline (speedup 1.0000x reference)
import functools

import jax
import jax.numpy as jnp
from jax import lax
from jax.experimental import pallas as pl
from jax.experimental.pallas import tpu as pltpu
from jax.experimental.pallas import tpu_sc as plsc

DIM = 64
BATCH = 16384


@functools.cache
def _build(B, D):
    info = plsc.get_sparse_core_info()
    NC, NS = info.num_cores, info.num_subcores
    NW = NC * NS
    assert B % (8 * NW) == 0 and D % info.num_lanes == 0
    b_per_w = B // NW
    mesh = plsc.VectorSubcoreMesh(core_axis_name="c", subcore_axis_name="s")

    @functools.partial(
        pl.kernel,
        mesh=mesh,
        compiler_params=pltpu.CompilerParams(use_tc_tiling_on_sc=False),
        out_type=(
            jax.ShapeDtypeStruct((B, D), jnp.float32),
            jax.ShapeDtypeStruct((B, D), jnp.float32),
        ),
        scratch_types=[
            pltpu.VMEM((b_per_w,), jnp.int32),
            pltpu.VMEM((b_per_w,), jnp.int32),
            pltpu.VMEM((b_per_w, D), jnp.float32),
            pltpu.VMEM((b_per_w, D), jnp.float32),
            pltpu.SemaphoreType.DMA,
            pltpu.SemaphoreType.DMA,
        ],
    )
    def k(uid_hbm, pid_hbm, ut_hbm, pt_hbm, u_out, p_out,
          uidx_v, pidx_v, urows_v, prows_v, sem_u, sem_p):
        wid = lax.axis_index("s") * NC + lax.axis_index("c")
        base = wid * b_per_w
        pltpu.sync_copy(uid_hbm.at[pl.ds(base, b_per_w)], uidx_v)
        pltpu.sync_copy(pid_hbm.at[pl.ds(base, b_per_w)], pidx_v)
        cu = pltpu.async_copy(ut_hbm.at[uidx_v], urows_v, sem_u)
        cp = pltpu.async_copy(pt_hbm.at[pidx_v], prows_v, sem_p)
        cu.wait()
        pltpu.sync_copy(urows_v, u_out.at[pl.ds(base, b_per_w)])
        cp.wait()
        pltpu.sync_copy(prows_v, p_out.at[pl.ds(base, b_per_w)])

    return k


def kernel(user_ids, product_ids, user_table, product_table):
    B, D = user_ids.shape[0], user_table.shape[1]
    k = _build(B, D)
    return k(user_ids.astype(jnp.int32), product_ids.astype(jnp.int32),
             user_table, product_table)


# speedup vs baseline: 2.2178x; 2.2178x over previous
import functools

import jax
import jax.numpy as jnp
from jax import lax
from jax.experimental import pallas as pl
from jax.experimental.pallas import tpu as pltpu
from jax.experimental.pallas import tpu_sc as plsc


@functools.cache
def _build(B, D, V):
    info = plsc.get_sparse_core_info()
    NC, NS, L = info.num_cores, info.num_subcores, info.num_lanes
    NW = NC * NS
    assert B % NW == 0 and D % L == 0 and V % 8 == 0
    bpw = B // NW
    NG = bpw // L
    mesh = plsc.VectorSubcoreMesh(core_axis_name="c", subcore_axis_name="s")

    @functools.partial(
        pl.kernel,
        mesh=mesh,
        out_type=(
            jax.ShapeDtypeStruct((B, D), jnp.float32),
            jax.ShapeDtypeStruct((B, D), jnp.float32),
        ),
        scratch_types=[
            pltpu.VMEM((bpw,), jnp.int32),
            pltpu.VMEM((bpw, D), jnp.float32),
            pltpu.SemaphoreType.DMA,
        ],
    )
    def k(uid, pid, ut3, pt3, u_out, p_out, ids_v, rows_v, sem):
        wid = lax.axis_index("s") * NC + lax.axis_index("c")
        base = wid * bpw

        def one_table(idx_hbm, tab3, out_hbm):
            pltpu.sync_copy(idx_hbm.at[pl.ds(base, bpw)], ids_v)

            def group_body(g, carry):
                ids_vec = ids_v[pl.ds(g * L, L)]
                copies = []
                for l in range(L):
                    id_s = ids_vec[l]
                    bid = lax.shift_right_logical(id_s, 3)
                    sub = id_s & 7
                    copies.append(pltpu.async_copy(
                        tab3.at[bid, sub], rows_v.at[g * L + l], sem))
                for cp in copies:
                    cp.wait()
                return carry

            lax.fori_loop(0, NG, group_body, 0)
            pltpu.sync_copy(rows_v, out_hbm.at[pl.ds(base, bpw)])

        one_table(uid, ut3, u_out)
        one_table(pid, pt3, p_out)

    return k


def kernel(user_ids, product_ids, user_table, product_table):
    V, D = user_table.shape
    B = user_ids.shape[0]
    k = _build(B, D, V)
    ut3 = user_table.reshape(V // 8, 8, D)
    pt3 = product_table.reshape(V // 8, 8, D)
    return k(user_ids.astype(jnp.int32), product_ids.astype(jnp.int32),
             ut3, pt3)
